# baseline (speedup 1.0000x reference)
import functools

import numpy as np
import jax
import jax.numpy as jnp
from jax import lax
from jax.experimental import pallas as pl
from jax.experimental.pallas import tpu as pltpu
from jax.experimental.pallas import tpu_sc as plsc

_VOCAB = 100000
_D = 768
_B = 4
_S = 2048
_N = _B * _S
_NW = 32
_SPW = _S // _NW
_T = 16
_QC = _SPW // _T
_NV = _D // 16
_RING = 5
_LEAD = 3


def _make_pe_np(seq_len, d_model):
    pos = np.arange(seq_len, dtype=np.float32)[:, None]
    div = np.exp(np.arange(0, d_model, 2, dtype=np.float32) * (-np.log(10000.0) / d_model))
    pe = np.zeros((seq_len, d_model), dtype=np.float32)
    pe[:, 0::2] = np.sin(pos * div)
    pe[:, 1::2] = np.cos(pos * div)
    return pe


_PE = jnp.asarray(_make_pe_np(_S, _D))

_mesh = plsc.VectorSubcoreMesh(core_axis_name="c", subcore_axis_name="s")


@functools.partial(
    pl.kernel,
    mesh=_mesh,
    out_type=jax.ShapeDtypeStruct((_N, _D), jnp.float32),
    scratch_types=[
        pltpu.VMEM((_B, _SPW), jnp.int32),
        pltpu.VMEM((_B, _SPW), jnp.int32),
        pltpu.VMEM((_RING, _T, _D), jnp.float32),
        pltpu.VMEM((_SPW, _D), jnp.float32),
        pltpu.VMEM((3, _D), jnp.float32),
    ] + [pltpu.SemaphoreType.DMA] * (2 * _RING + 1),
)
def _emb_kernel(seq_hbm, segl_hbm, tok_tab, seg_tab, pe_hbm, out_hbm,
                idx_v, sidx_v, tok_v, pe_v, segtab_v, *sems):
    gsems = sems[:_RING]
    osems = sems[_RING:2 * _RING]
    psem = sems[2 * _RING]
    wid = lax.axis_index("s") * 2 + lax.axis_index("c")
    s0 = wid * _SPW

    idx_cps = [pltpu.async_copy(seq_hbm.at[b, pl.ds(s0, _SPW)], idx_v.at[b], psem)
               for b in range(_B)]
    sidx_cps = [pltpu.async_copy(segl_hbm.at[b, pl.ds(s0, _SPW)], sidx_v.at[b], psem)
                for b in range(_B)]
    for cp in idx_cps:
        cp.wait()

    chunks = [(b, q) for b in range(_B) for q in range(_QC)]
    NK = len(chunks)

    def start_gather(m):
        b, q = chunks[m]
        return pltpu.async_copy(
            tok_tab.at[idx_v.at[b, pl.ds(q * _T, _T)]],
            tok_v.at[m % _RING], gsems[m % _RING])

    gcp = [None] * NK
    ocp = [None] * NK
    for m in range(_LEAD):
        gcp[m] = start_gather(m)

    cp_pe = pltpu.async_copy(pe_hbm.at[pl.ds(s0, _SPW)], pe_v, psem)
    pltpu.sync_copy(seg_tab, segtab_v)
    for cp in sidx_cps:
        cp.wait()
    cp_pe.wait()

    for k in range(NK):
        b, q = chunks[k]
        p = k % _RING
        off = b * _S + s0 + q * _T
        m = k + _LEAD
        if m < NK:
            if m - _RING >= 0:
                ocp[m - _RING].wait()
            gcp[m] = start_gather(m)
        gcp[k].wait()

        svec = sidx_v[b, pl.ds(q * _T, 16)]
        c1 = [(svec[l] >= 1).astype(jnp.float32) for l in range(16)]
        c2 = [(svec[l] == 2).astype(jnp.float32) for l in range(16)]

        @plsc.parallel_loop(0, _NV)
        def jbody(j, _p=p, _q=q, _c1=c1, _c2=c2):
            sl = pl.ds(j * 16, 16)
            s0v = segtab_v[0, sl]
            d1 = segtab_v[1, sl] - s0v
            d2 = segtab_v[2, sl] - segtab_v[1, sl]
            for l in range(16):
                pr = _q * _T + l
                tok_v[_p, l, sl] = (tok_v[_p, l, sl] + pe_v[pr, sl] + s0v
                                    + _c1[l] * d1 + _c2[l] * d2)

        ocp[k] = pltpu.async_copy(tok_v.at[p], out_hbm.at[pl.ds(off, _T)], osems[p])

    for k in range(NK - _RING, NK):
        if ocp[k] is not None:
            ocp[k].wait()


def kernel(sequence, segment_label, token_table, seg_table):
    out = _emb_kernel(sequence, segment_label, token_table, seg_table, _PE)
    return out.reshape(_B, _S, _D)


# speedup vs baseline: 1.0861x; 1.0861x over previous
import functools

import numpy as np
import jax
import jax.numpy as jnp
from jax import lax
from jax.experimental import pallas as pl
from jax.experimental.pallas import tpu as pltpu
from jax.experimental.pallas import tpu_sc as plsc

_VOCAB = 100000
_D = 768
_B = 4
_S = 2048
_N = _B * _S
_NW = 32
_SPW = _S // _NW
_T = 16
_QC = _SPW // _T
_NV = _D // 16


def _make_pe_np(seq_len, d_model):
    pos = np.arange(seq_len, dtype=np.float32)[:, None]
    div = np.exp(np.arange(0, d_model, 2, dtype=np.float32) * (-np.log(10000.0) / d_model))
    pe = np.zeros((seq_len, d_model), dtype=np.float32)
    pe[:, 0::2] = np.sin(pos * div)
    pe[:, 1::2] = np.cos(pos * div)
    return pe


_PE = jnp.asarray(_make_pe_np(_S, _D))

_mesh = plsc.VectorSubcoreMesh(core_axis_name="c", subcore_axis_name="s")


@functools.partial(
    pl.kernel,
    mesh=_mesh,
    out_type=jax.ShapeDtypeStruct((_N, _D), jnp.float32),
    scratch_types=[
        pltpu.VMEM((_B, _SPW), jnp.int32),
        pltpu.VMEM((_B, _SPW), jnp.int32),
        pltpu.VMEM((_QC, _T, _D), jnp.float32),
        pltpu.VMEM((_SPW, _D), jnp.float32),
        pltpu.VMEM((3, _D), jnp.float32),
    ] + [pltpu.SemaphoreType.DMA] * (2 * _QC + 1),
)
def _emb_kernel(seq_hbm, segl_hbm, tok_tab, seg_tab, pe_hbm, out_hbm,
                idx_v, sidx_v, tok_v, pe_v, segtab_v, *sems):
    gsems = sems[:_QC]
    osems = sems[_QC:2 * _QC]
    psem = sems[2 * _QC]
    wid = lax.axis_index("s") * 2 + lax.axis_index("c")
    s0 = wid * _SPW

    idx_cps = [pltpu.async_copy(seq_hbm.at[b, pl.ds(s0, _SPW)], idx_v.at[b], psem)
               for b in range(_B)]
    sidx_cps = [pltpu.async_copy(segl_hbm.at[b, pl.ds(s0, _SPW)], sidx_v.at[b], psem)
                for b in range(_B)]
    for cp in idx_cps:
        cp.wait()

    def start_gather(b, q):
        return pltpu.async_copy(
            tok_tab.at[idx_v.at[b, pl.ds(q * _T, _T)]],
            tok_v.at[q], gsems[q])

    def wait_gather(q):
        pltpu.make_async_copy(tok_tab.at[idx_v.at[0, pl.ds(q * _T, _T)]],
                              tok_v.at[q], gsems[q]).wait()

    def wait_write(q):
        pltpu.make_async_copy(tok_v.at[q], out_hbm.at[pl.ds(s0, _T)],
                              osems[q]).wait()

    start_gather(0, 0)
    start_gather(0, 1)

    cp_pe = pltpu.async_copy(pe_hbm.at[pl.ds(s0, _SPW)], pe_v, psem)
    pltpu.sync_copy(seg_tab, segtab_v)
    for cp in sidx_cps:
        cp.wait()
    cp_pe.wait()

    def batch_body(b, carry):
        for q in range(_QC):
            if q < 2:
                @pl.when(b >= 1)
                def _():
                    wait_write(q + 2)
                start_gather(b, q + 2)
            else:
                @pl.when(b < _B - 1)
                def _():
                    wait_write(q - 2)
                    start_gather(b + 1, q - 2)

            wait_gather(q)

            svec = sidx_v[b, pl.ds(q * _T, 16)]
            c1 = [(svec[l] >= 1).astype(jnp.float32) for l in range(16)]
            c2 = [(svec[l] == 2).astype(jnp.float32) for l in range(16)]

            @plsc.parallel_loop(0, _NV)
            def jbody(j, _q=q, _c1=c1, _c2=c2):
                sl = pl.ds(j * 16, 16)
                s0v = segtab_v[0, sl]
                d1 = segtab_v[1, sl] - s0v
                d2 = segtab_v[2, sl] - segtab_v[1, sl]
                for l in range(16):
                    pr = _q * _T + l
                    tok_v[_q, l, sl] = (tok_v[_q, l, sl] + pe_v[pr, sl] + s0v
                                        + _c1[l] * d1 + _c2[l] * d2)

            off = b * _S + s0 + q * _T
            pltpu.async_copy(tok_v.at[q], out_hbm.at[pl.ds(off, _T)], osems[q])
        return carry

    lax.fori_loop(0, _B, batch_body, 0)

    for q in range(_QC):
        wait_write(q)


def kernel(sequence, segment_label, token_table, seg_table):
    out = _emb_kernel(sequence, segment_label, token_table, seg_table, _PE)
    return out.reshape(_B, _S, _D)
